# baseline (speedup 1.0000x reference)
import functools

import jax
import jax.numpy as jnp
from jax import lax
from jax.experimental import pallas as pl
from jax.experimental.pallas import tpu as pltpu
from jax.experimental.pallas import tpu_sc as plsc

N = 10000
E = 320000
NF = 16
EF = 16
IN = NF
OUT = 2 * NF
LEAK = 0.1
EPS = 1e-5

NC = 2
NS = 16
NW = NC * NS
EW = E // NW

@functools.lru_cache(maxsize=None)
def _sc_mesh():
    return plsc.VectorSubcoreMesh(core_axis_name="c", subcore_axis_name="s",
                                  num_cores=NC, num_subcores=NS)


def _lrelu(v):
    return jnp.where(v >= 0, v, LEAK * v)


EQ = E // 8
_BE = 1600


def _fold16(s):
    for w in (64, 32, 16):
        s = s[:, 0:w] + s[:, w:2 * w]
    return s


def _stats_body(e_ref, sum_ref, sq_ref):
    i = pl.program_id(0)
    eb = e_ref[...]
    s = _fold16(jnp.sum(eb, axis=0, keepdims=True))
    q = _fold16(jnp.sum(eb * eb, axis=0, keepdims=True))

    @pl.when(i == 0)
    def _():
        sum_ref[...] = s
        sq_ref[...] = q

    @pl.when(i > 0)
    def _():
        sum_ref[...] += s
        sq_ref[...] += q


def _stats(ep):
    return pl.pallas_call(
        _stats_body,
        grid=(EQ // _BE,),
        in_specs=[pl.BlockSpec((_BE, 128), lambda i: (i, 0))],
        out_specs=[pl.BlockSpec((1, EF), lambda i: (0, 0)),
                   pl.BlockSpec((1, EF), lambda i: (0, 0))],
        out_shape=[jax.ShapeDtypeStruct((1, EF), jnp.float32),
                   jax.ShapeDtypeStruct((1, EF), jnp.float32)],
    )(ep)


def _prep_body(x_ref, esum_ref, esq_ref, gn_ref, bn_ref, ge_ref, be_ref,
               rw_ref, rb_ref,
               root_ref, xsc_ref, xsh_ref, esc_ref, esh_ref):
    x = x_ref[...]
    xm = jnp.mean(x, axis=0, keepdims=True)
    xv = jnp.mean(x * x, axis=0, keepdims=True) - xm * xm
    xsc = gn_ref[...] * lax.rsqrt(xv + EPS)
    xsh = bn_ref[...] - xm * xsc
    em = esum_ref[...] * (1.0 / E)
    ev = esq_ref[...] * (1.0 / E) - em * em
    esc = ge_ref[...] * lax.rsqrt(ev + EPS)
    esh = be_ref[...] - em * esc
    xb = x * xsc + xsh
    root_ref[...] = (jnp.dot(xb, rw_ref[...], preferred_element_type=jnp.float32)
                     + rb_ref[...])
    xsc_ref[...] = xsc
    xsh_ref[...] = xsh
    esc_ref[...] = esc
    esh_ref[...] = esh


def _prep(x, esum, esq, gn, bn, ge, be, rw, rb):
    v16 = jax.ShapeDtypeStruct((1, NF), jnp.float32)
    return pl.pallas_call(
        _prep_body,
        out_shape=[jax.ShapeDtypeStruct((N, OUT), jnp.float32), v16, v16, v16, v16],
    )(x, esum, esq, gn, bn, ge, be, rw, rb)


_C2 = 2000


def _gather_x_body(x_hbm, src_hbm, out_hbm, idx_v, rows_v, sem):
    wid = lax.axis_index("s") * NC + lax.axis_index("c")
    base = wid * EW

    def step(j, carry):
        off = pl.multiple_of(base + j * _C2, 8)
        pltpu.sync_copy(src_hbm.at[pl.ds(off, _C2)], idx_v)
        pltpu.async_copy(x_hbm.at[idx_v], rows_v, sem).wait()
        pltpu.sync_copy(rows_v, out_hbm.at[pl.ds(off, _C2)])
        return carry

    lax.fori_loop(0, EW // _C2, step, 0)


@functools.lru_cache(maxsize=None)
def _gather_x():
    return pl.kernel(
        _gather_x_body,
        out_type=jax.ShapeDtypeStruct((E, NF), jnp.float32),
        mesh=_sc_mesh(),
        compiler_params=pltpu.CompilerParams(use_tc_tiling_on_sc=False),
        scratch_types=[pltpu.VMEM((_C2,), jnp.int32),
                       pltpu.VMEM((_C2, NF), jnp.float32),
                       pltpu.SemaphoreType.DMA])


_BQ = 400


def _msg_body(ep_ref, xsp_ref, w1b_ref, b1b_ref, w2b_ref, b2b_ref,
              rxb_ref, shx_ref, sum_ref, outa_ref, outb_ref):
    f32 = jnp.float32
    bf16 = jnp.bfloat16
    hP = _lrelu(jnp.dot(ep_ref[...].astype(bf16), w1b_ref[...],
                        preferred_element_type=f32)
                + b1b_ref[...])
    zP = _lrelu(jnp.dot(hP.astype(bf16), w2b_ref[...],
                        preferred_element_type=f32)
                + b2b_ref[...])
    xeP = (jnp.dot(xsp_ref[...].astype(bf16), rxb_ref[...],
                   preferred_element_type=f32)
           + shx_ref[...])
    p = xeP * zP
    m = jnp.dot(p.astype(bf16), sum_ref[...], preferred_element_type=f32)
    outa_ref[...] = m[:, 0:128]
    outb_ref[...] = m[:, 128:256]


def _msg(ep, xsp, w1b, b1b, w2b, b2b, rxb, shx, sumb):
    c = lambda i: (0, 0)
    return pl.pallas_call(
        _msg_body,
        grid=(EQ // _BQ,),
        in_specs=[pl.BlockSpec((_BQ, 128), lambda i: (i, 0)),
                  pl.BlockSpec((_BQ, 128), lambda i: (i, 0)),
                  pl.BlockSpec((128, 128), c), pl.BlockSpec((1, 128), c),
                  pl.BlockSpec((128, 8 * IN * OUT), c),
                  pl.BlockSpec((1, 8 * IN * OUT), c),
                  pl.BlockSpec((128, 8 * IN * OUT), c),
                  pl.BlockSpec((1, 8 * IN * OUT), c),
                  pl.BlockSpec((8 * IN * OUT, 256), c)],
        out_specs=[pl.BlockSpec((_BQ, 128), lambda i: (i, 0)),
                   pl.BlockSpec((_BQ, 128), lambda i: (i, 0))],
        out_shape=[jax.ShapeDtypeStruct((EQ, 128), jnp.float32),
                   jax.ShapeDtypeStruct((EQ, 128), jnp.float32)],
    )(ep, xsp, w1b, b1b, w2b, b2b, rxb, shx, sumb)


EH = E // 2
EHW = EH // NW
_C4 = 1000


def _scatter_msg_body(ma_hbm, mb_hbm, da_hbm, db_hbm, zero_hbm, out_hbm,
                      idx_v, upd_v, acc_sh):
    cid = lax.axis_index("c")
    sid = lax.axis_index("s")

    @pl.when(sid == 0)
    def _():
        pltpu.sync_copy(zero_hbm, acc_sh)

    plsc.subcore_barrier()
    base = (sid * NC + cid) * EHW

    def step(j, carry):
        off = pl.multiple_of(base + j * _C4, 8)
        pltpu.sync_copy(da_hbm.at[pl.ds(off, _C4)], idx_v)
        pltpu.sync_copy(ma_hbm.at[pl.ds(off, _C4)], upd_v)
        pltpu.sync_copy(upd_v, acc_sh.at[idx_v], add=True)
        pltpu.sync_copy(db_hbm.at[pl.ds(off, _C4)], idx_v)
        pltpu.sync_copy(mb_hbm.at[pl.ds(off, _C4)], upd_v)
        pltpu.sync_copy(upd_v, acc_sh.at[idx_v], add=True)
        return carry

    lax.fori_loop(0, EHW // _C4, step, 0)
    plsc.subcore_barrier()

    @pl.when(sid == 0)
    def _():
        pltpu.sync_copy(acc_sh, out_hbm.at[cid])


@functools.lru_cache(maxsize=None)
def _scatter_msg():
    return pl.kernel(
        _scatter_msg_body,
        out_type=jax.ShapeDtypeStruct((NC, N, OUT), jnp.float32),
        mesh=_sc_mesh(),
        compiler_params=pltpu.CompilerParams(use_tc_tiling_on_sc=False),
        scratch_types=[pltpu.VMEM((_C4,), jnp.int32),
                       pltpu.VMEM((_C4, OUT), jnp.float32),
                       pltpu.VMEM_SHARED((N, OUT), jnp.float32)])


def _blk(m):
    return jnp.kron(jnp.eye(8, dtype=m.dtype), m)


def _combine_body(p0_ref, p1_ref, root_ref, w1a_ref, w1b_ref, out_ref):
    f32 = jnp.float32
    node = p0_ref[...] + p1_ref[...] + root_ref[...]
    a = jnp.dot(node, w1a_ref[...], preferred_element_type=f32)
    b = jnp.dot(node, w1b_ref[...], preferred_element_type=f32)
    out_ref[...] = jnp.concatenate([a, b], axis=1)


def _combine(p0, p1, root, w1a, w1b):
    return pl.pallas_call(
        _combine_body,
        out_shape=jax.ShapeDtypeStruct((N, 128), jnp.float32),
    )(p0, p1, root, w1a, w1b)


_C6 = 1000


def _gather_n_body(nodes_hbm, src_hbm, dst_hbm, osrc_hbm, odst_hbm,
                   idx_s, idx_d, rows_v, sem):
    wid = lax.axis_index("s") * NC + lax.axis_index("c")
    base = wid * EW

    def step(j, carry):
        off = pl.multiple_of(base + j * _C6, 8)
        pltpu.sync_copy(src_hbm.at[pl.ds(off, _C6)], idx_s)
        pltpu.sync_copy(dst_hbm.at[pl.ds(off, _C6)], idx_d)
        pltpu.async_copy(nodes_hbm.at[idx_s], rows_v, sem).wait()
        pltpu.sync_copy(rows_v, osrc_hbm.at[pl.ds(off, _C6)])
        pltpu.async_copy(nodes_hbm.at[idx_d], rows_v, sem).wait()
        pltpu.sync_copy(rows_v, odst_hbm.at[pl.ds(off, _C6)])
        return carry

    lax.fori_loop(0, EW // _C6, step, 0)


@functools.lru_cache(maxsize=None)
def _gather_n():
    return pl.kernel(
        _gather_n_body,
        out_type=(jax.ShapeDtypeStruct((E, 128), jnp.float32),
                  jax.ShapeDtypeStruct((E, 128), jnp.float32)),
        mesh=_sc_mesh(),
        compiler_params=pltpu.CompilerParams(use_tc_tiling_on_sc=False),
        scratch_types=[pltpu.VMEM((_C6,), jnp.int32),
                       pltpu.VMEM((_C6,), jnp.int32),
                       pltpu.VMEM((_C6, 128), jnp.float32),
                       pltpu.SemaphoreType.DMA])


_B7 = 3200


def _mlp_body(e_ref, gs_ref, gd_ref, esc_ref, esh_ref,
              w1c_ref, b1_ref, w2_ref, b2_ref, w3_ref, b3_ref,
              w4_ref, b4_ref, w5_ref, b5_ref, out_ref):
    f32 = jnp.float32
    eb = e_ref[...] * esc_ref[...] + esh_ref[...]
    z = (gs_ref[:, 0:64] + gd_ref[:, 64:128]
         + jnp.dot(eb, w1c_ref[...], preferred_element_type=f32)
         + b1_ref[...])
    z = _lrelu(z)
    z = _lrelu(jnp.dot(z, w2_ref[...], preferred_element_type=f32) + b2_ref[...])
    z = _lrelu(jnp.dot(z, w3_ref[...], preferred_element_type=f32) + b3_ref[...])
    z = _lrelu(jnp.dot(z, w4_ref[...], preferred_element_type=f32) + b4_ref[...])
    out_ref[...] = jnp.dot(z, w5_ref[...], preferred_element_type=f32) + b5_ref[...]


def _mlp(e, gs, gd, esc, esh, w1c, b1, w2, b2, w3, b3, w4, b4, w5, b5):
    c = lambda i: (0, 0)
    return pl.pallas_call(
        _mlp_body,
        grid=(E // _B7,),
        in_specs=[pl.BlockSpec((_B7, EF), lambda i: (i, 0)),
                  pl.BlockSpec((_B7, 128), lambda i: (i, 0)),
                  pl.BlockSpec((_B7, 128), lambda i: (i, 0)),
                  pl.BlockSpec((1, EF), c), pl.BlockSpec((1, EF), c),
                  pl.BlockSpec((EF, 64), c), pl.BlockSpec((1, 64), c),
                  pl.BlockSpec((64, 32), c), pl.BlockSpec((1, 32), c),
                  pl.BlockSpec((32, 16), c), pl.BlockSpec((1, 16), c),
                  pl.BlockSpec((16, 8), c), pl.BlockSpec((1, 8), c),
                  pl.BlockSpec((8, 2), c), pl.BlockSpec((1, 2), c)],
        out_specs=pl.BlockSpec((_B7, 2), lambda i: (i, 0)),
        out_shape=jax.ShapeDtypeStruct((E, 2), jnp.float32),
    )(e, gs, gd, esc, esh, w1c, b1, w2, b2, w3, b3, w4, b4, w5, b5)


def kernel(x, edge_index, e, xbatch, bn_node_gamma, bn_node_beta,
           bn_edge_gamma, bn_edge_beta, nn_w1, nn_b1, nn_w2, nn_b2,
           conv_root, conv_bias, ep_w1, ep_b1, ep_w2, ep_b2, ep_w3, ep_b3,
           ep_w4, ep_b4, ep_w5, ep_b5):
    src = edge_index[0]
    dst = edge_index[1]
    r2 = lambda v: v.reshape(1, -1)
    ep = e.reshape(EQ, 128)

    esum, esq = _stats(ep)
    root, xsc, xsh, esc, esh = _prep(
        x, esum, esq, r2(bn_node_gamma), r2(bn_node_beta),
        r2(bn_edge_gamma), r2(bn_edge_beta), conv_root, r2(conv_bias))

    xsp = _gather_x()(x, src).reshape(EQ, 128)

    rep = jnp.repeat(jnp.eye(NF, dtype=jnp.float32), OUT, axis=1)
    w1b = _blk(esc.reshape(EF, 1) * nn_w1).astype(jnp.bfloat16)
    b1b = jnp.tile(esh @ nn_w1 + nn_b1.reshape(1, IN), (1, 8))
    w2b = _blk(nn_w2).astype(jnp.bfloat16)
    b2b = jnp.tile(nn_b2.reshape(1, IN * OUT), (1, 8))
    rxb = _blk(xsc.reshape(NF, 1) * rep).astype(jnp.bfloat16)
    shx = jnp.tile(xsh @ rep, (1, 8))
    sumb = _blk(jnp.tile(jnp.eye(OUT, dtype=jnp.float32),
                         (IN, 1))).astype(jnp.bfloat16)
    msga, msgb = _msg(ep, xsp, w1b, b1b, w2b, b2b, rxb, shx, sumb)

    dst8 = dst.reshape(EQ, 8)
    dsta = dst8[:, 0:4].reshape(-1)
    dstb = dst8[:, 4:8].reshape(-1)
    zeros = jnp.zeros((N, OUT), jnp.float32)
    partials = _scatter_msg()(msga.reshape(EH, OUT), msgb.reshape(EH, OUT),
                              dsta, dstb, zeros)
    tbl = _combine(partials[0], partials[1], root,
                   ep_w1[0:OUT], ep_w1[OUT:2 * OUT])

    gs, gd = _gather_n()(tbl, src, dst)
    return _mlp(e, gs, gd, esc, esh, ep_w1[2 * OUT:2 * OUT + EF], r2(ep_b1),
                ep_w2, r2(ep_b2), ep_w3, r2(ep_b3), ep_w4, r2(ep_b4),
                ep_w5, r2(ep_b5))


# speedup vs baseline: 1.0445x; 1.0445x over previous
import functools

import jax
import jax.numpy as jnp
from jax import lax
from jax.experimental import pallas as pl
from jax.experimental.pallas import tpu as pltpu
from jax.experimental.pallas import tpu_sc as plsc

N = 10000
E = 320000
NF = 16
EF = 16
IN = NF
OUT = 2 * NF
LEAK = 0.1
EPS = 1e-5

NC = 2
NS = 16
NW = NC * NS
EW = E // NW

@functools.lru_cache(maxsize=None)
def _sc_mesh():
    return plsc.VectorSubcoreMesh(core_axis_name="c", subcore_axis_name="s",
                                  num_cores=NC, num_subcores=NS)


def _lrelu(v):
    return jnp.where(v >= 0, v, LEAK * v)


EQ = E // 8
_BE = 1600


def _fold16(s):
    for w in (64, 32, 16):
        s = s[:, 0:w] + s[:, w:2 * w]
    return s


def _stats_body(e_ref, sum_ref, sq_ref):
    i = pl.program_id(0)
    eb = e_ref[...]
    s = _fold16(jnp.sum(eb, axis=0, keepdims=True))
    q = _fold16(jnp.sum(eb * eb, axis=0, keepdims=True))

    @pl.when(i == 0)
    def _():
        sum_ref[...] = s
        sq_ref[...] = q

    @pl.when(i > 0)
    def _():
        sum_ref[...] += s
        sq_ref[...] += q


def _stats(ep):
    return pl.pallas_call(
        _stats_body,
        grid=(EQ // _BE,),
        in_specs=[pl.BlockSpec((_BE, 128), lambda i: (i, 0))],
        out_specs=[pl.BlockSpec((1, EF), lambda i: (0, 0)),
                   pl.BlockSpec((1, EF), lambda i: (0, 0))],
        out_shape=[jax.ShapeDtypeStruct((1, EF), jnp.float32),
                   jax.ShapeDtypeStruct((1, EF), jnp.float32)],
    )(ep)


def _prep_body(x_ref, esum_ref, esq_ref, gn_ref, bn_ref, ge_ref, be_ref,
               rw_ref, rb_ref,
               root_ref, xsc_ref, xsh_ref, esc_ref, esh_ref):
    x = x_ref[...]
    xm = jnp.mean(x, axis=0, keepdims=True)
    xv = jnp.mean(x * x, axis=0, keepdims=True) - xm * xm
    xsc = gn_ref[...] * lax.rsqrt(xv + EPS)
    xsh = bn_ref[...] - xm * xsc
    em = esum_ref[...] * (1.0 / E)
    ev = esq_ref[...] * (1.0 / E) - em * em
    esc = ge_ref[...] * lax.rsqrt(ev + EPS)
    esh = be_ref[...] - em * esc
    xb = x * xsc + xsh
    root_ref[...] = (jnp.dot(xb, rw_ref[...], preferred_element_type=jnp.float32)
                     + rb_ref[...])
    xsc_ref[...] = xsc
    xsh_ref[...] = xsh
    esc_ref[...] = esc
    esh_ref[...] = esh


def _prep(x, esum, esq, gn, bn, ge, be, rw, rb):
    v16 = jax.ShapeDtypeStruct((1, NF), jnp.float32)
    return pl.pallas_call(
        _prep_body,
        out_shape=[jax.ShapeDtypeStruct((N, OUT), jnp.float32), v16, v16, v16, v16],
    )(x, esum, esq, gn, bn, ge, be, rw, rb)


_C2 = 2000


def _gather_x_body(x_hbm, src_hbm, out_hbm, idx_v, rows_v, sem):
    wid = lax.axis_index("s") * NC + lax.axis_index("c")
    base = wid * EW

    def step(j, carry):
        off = pl.multiple_of(base + j * _C2, 8)
        pltpu.sync_copy(src_hbm.at[pl.ds(off, _C2)], idx_v)
        pltpu.async_copy(x_hbm.at[idx_v], rows_v, sem).wait()
        pltpu.sync_copy(rows_v, out_hbm.at[pl.ds(off, _C2)])
        return carry

    lax.fori_loop(0, EW // _C2, step, 0)


@functools.lru_cache(maxsize=None)
def _gather_x():
    return pl.kernel(
        _gather_x_body,
        out_type=jax.ShapeDtypeStruct((E, NF), jnp.float32),
        mesh=_sc_mesh(),
        compiler_params=pltpu.CompilerParams(use_tc_tiling_on_sc=False),
        scratch_types=[pltpu.VMEM((_C2,), jnp.int32),
                       pltpu.VMEM((_C2, NF), jnp.float32),
                       pltpu.SemaphoreType.DMA])


_BQ = 400


def _msg_body(ep_ref, xsp_ref, w1b_ref, b1b_ref, w2b_ref, b2b_ref,
              rxb_ref, shx_ref, outa_ref, outb_ref):
    f32 = jnp.float32
    bf16 = jnp.bfloat16
    hP = _lrelu(jnp.dot(ep_ref[...].astype(bf16), w1b_ref[...],
                        preferred_element_type=f32)
                + b1b_ref[...])
    zP = _lrelu(jnp.dot(hP.astype(bf16), w2b_ref[...],
                        preferred_element_type=f32)
                + b2b_ref[...])
    xeP = (jnp.dot(xsp_ref[...].astype(bf16), rxb_ref[...],
                   preferred_element_type=f32)
           + shx_ref[...])
    p = xeP * zP
    groups = []
    for g in range(8):
        q = p[:, g * 512:g * 512 + 256] + p[:, g * 512 + 256:g * 512 + 512]
        for w in (128, 64, 32):
            q = q[:, 0:w] + q[:, w:2 * w]
        groups.append(q)
    outa_ref[...] = jnp.concatenate(groups[0:4], axis=1)
    outb_ref[...] = jnp.concatenate(groups[4:8], axis=1)


def _msg(ep, xsp, w1b, b1b, w2b, b2b, rxb, shx):
    c = lambda i: (0, 0)
    return pl.pallas_call(
        _msg_body,
        grid=(EQ // _BQ,),
        in_specs=[pl.BlockSpec((_BQ, 128), lambda i: (i, 0)),
                  pl.BlockSpec((_BQ, 128), lambda i: (i, 0)),
                  pl.BlockSpec((128, 128), c), pl.BlockSpec((1, 128), c),
                  pl.BlockSpec((128, 8 * IN * OUT), c),
                  pl.BlockSpec((1, 8 * IN * OUT), c),
                  pl.BlockSpec((128, 8 * IN * OUT), c),
                  pl.BlockSpec((1, 8 * IN * OUT), c)],
        out_specs=[pl.BlockSpec((_BQ, 128), lambda i: (i, 0)),
                   pl.BlockSpec((_BQ, 128), lambda i: (i, 0))],
        out_shape=[jax.ShapeDtypeStruct((EQ, 128), jnp.float32),
                   jax.ShapeDtypeStruct((EQ, 128), jnp.float32)],
    )(ep, xsp, w1b, b1b, w2b, b2b, rxb, shx)


EH = E // 2
EHW = EH // NW
_C4 = 1000


def _scatter_msg_body(ma_hbm, mb_hbm, da_hbm, db_hbm, zero_hbm, out_hbm,
                      idx_v, upd_v, acc_sh):
    cid = lax.axis_index("c")
    sid = lax.axis_index("s")

    @pl.when(sid == 0)
    def _():
        pltpu.sync_copy(zero_hbm, acc_sh)

    plsc.subcore_barrier()
    base = (sid * NC + cid) * EHW

    def step(j, carry):
        off = pl.multiple_of(base + j * _C4, 8)
        pltpu.sync_copy(da_hbm.at[pl.ds(off, _C4)], idx_v)
        pltpu.sync_copy(ma_hbm.at[pl.ds(off, _C4)], upd_v)
        pltpu.sync_copy(upd_v, acc_sh.at[idx_v], add=True)
        pltpu.sync_copy(db_hbm.at[pl.ds(off, _C4)], idx_v)
        pltpu.sync_copy(mb_hbm.at[pl.ds(off, _C4)], upd_v)
        pltpu.sync_copy(upd_v, acc_sh.at[idx_v], add=True)
        return carry

    lax.fori_loop(0, EHW // _C4, step, 0)
    plsc.subcore_barrier()

    @pl.when(sid == 0)
    def _():
        pltpu.sync_copy(acc_sh, out_hbm.at[cid])


@functools.lru_cache(maxsize=None)
def _scatter_msg():
    return pl.kernel(
        _scatter_msg_body,
        out_type=jax.ShapeDtypeStruct((NC, N, OUT), jnp.float32),
        mesh=_sc_mesh(),
        compiler_params=pltpu.CompilerParams(use_tc_tiling_on_sc=False),
        scratch_types=[pltpu.VMEM((_C4,), jnp.int32),
                       pltpu.VMEM((_C4, OUT), jnp.float32),
                       pltpu.VMEM_SHARED((N, OUT), jnp.float32)])


def _blk(m):
    return jnp.kron(jnp.eye(8, dtype=m.dtype), m)


def _combine_body(p0_ref, p1_ref, root_ref, w1a_ref, w1b_ref, out_ref):
    f32 = jnp.float32
    node = p0_ref[...] + p1_ref[...] + root_ref[...]
    a = jnp.dot(node, w1a_ref[...], preferred_element_type=f32)
    b = jnp.dot(node, w1b_ref[...], preferred_element_type=f32)
    out_ref[...] = jnp.concatenate([a, b], axis=1)


def _combine(p0, p1, root, w1a, w1b):
    return pl.pallas_call(
        _combine_body,
        out_shape=jax.ShapeDtypeStruct((N, 128), jnp.float32),
    )(p0, p1, root, w1a, w1b)


_C6 = 1000


def _gather_n_body(nodes_hbm, src_hbm, dst_hbm, osrc_hbm, odst_hbm,
                   idx_s, idx_d, rows_v, sem):
    wid = lax.axis_index("s") * NC + lax.axis_index("c")
    base = wid * EW

    def step(j, carry):
        off = pl.multiple_of(base + j * _C6, 8)
        pltpu.sync_copy(src_hbm.at[pl.ds(off, _C6)], idx_s)
        pltpu.sync_copy(dst_hbm.at[pl.ds(off, _C6)], idx_d)
        pltpu.async_copy(nodes_hbm.at[idx_s], rows_v, sem).wait()
        pltpu.sync_copy(rows_v, osrc_hbm.at[pl.ds(off, _C6)])
        pltpu.async_copy(nodes_hbm.at[idx_d], rows_v, sem).wait()
        pltpu.sync_copy(rows_v, odst_hbm.at[pl.ds(off, _C6)])
        return carry

    lax.fori_loop(0, EW // _C6, step, 0)


@functools.lru_cache(maxsize=None)
def _gather_n():
    return pl.kernel(
        _gather_n_body,
        out_type=(jax.ShapeDtypeStruct((E, 128), jnp.float32),
                  jax.ShapeDtypeStruct((E, 128), jnp.float32)),
        mesh=_sc_mesh(),
        compiler_params=pltpu.CompilerParams(use_tc_tiling_on_sc=False),
        scratch_types=[pltpu.VMEM((_C6,), jnp.int32),
                       pltpu.VMEM((_C6,), jnp.int32),
                       pltpu.VMEM((_C6, 128), jnp.float32),
                       pltpu.SemaphoreType.DMA])


_B7 = 3200


def _mlp_body(e_ref, gs_ref, gd_ref, esc_ref, esh_ref,
              w1c_ref, b1_ref, w2_ref, b2_ref, w3_ref, b3_ref,
              w4_ref, b4_ref, w5_ref, b5_ref, out_ref):
    f32 = jnp.float32
    eb = e_ref[...] * esc_ref[...] + esh_ref[...]
    z = (gs_ref[:, 0:64] + gd_ref[:, 64:128]
         + jnp.dot(eb, w1c_ref[...], preferred_element_type=f32)
         + b1_ref[...])
    z = _lrelu(z)
    z = _lrelu(jnp.dot(z, w2_ref[...], preferred_element_type=f32) + b2_ref[...])
    z = _lrelu(jnp.dot(z, w3_ref[...], preferred_element_type=f32) + b3_ref[...])
    z = _lrelu(jnp.dot(z, w4_ref[...], preferred_element_type=f32) + b4_ref[...])
    out_ref[...] = jnp.dot(z, w5_ref[...], preferred_element_type=f32) + b5_ref[...]


def _mlp(e, gs, gd, esc, esh, w1c, b1, w2, b2, w3, b3, w4, b4, w5, b5):
    c = lambda i: (0, 0)
    return pl.pallas_call(
        _mlp_body,
        grid=(E // _B7,),
        in_specs=[pl.BlockSpec((_B7, EF), lambda i: (i, 0)),
                  pl.BlockSpec((_B7, 128), lambda i: (i, 0)),
                  pl.BlockSpec((_B7, 128), lambda i: (i, 0)),
                  pl.BlockSpec((1, EF), c), pl.BlockSpec((1, EF), c),
                  pl.BlockSpec((EF, 64), c), pl.BlockSpec((1, 64), c),
                  pl.BlockSpec((64, 32), c), pl.BlockSpec((1, 32), c),
                  pl.BlockSpec((32, 16), c), pl.BlockSpec((1, 16), c),
                  pl.BlockSpec((16, 8), c), pl.BlockSpec((1, 8), c),
                  pl.BlockSpec((8, 2), c), pl.BlockSpec((1, 2), c)],
        out_specs=pl.BlockSpec((_B7, 2), lambda i: (i, 0)),
        out_shape=jax.ShapeDtypeStruct((E, 2), jnp.float32),
    )(e, gs, gd, esc, esh, w1c, b1, w2, b2, w3, b3, w4, b4, w5, b5)


def kernel(x, edge_index, e, xbatch, bn_node_gamma, bn_node_beta,
           bn_edge_gamma, bn_edge_beta, nn_w1, nn_b1, nn_w2, nn_b2,
           conv_root, conv_bias, ep_w1, ep_b1, ep_w2, ep_b2, ep_w3, ep_b3,
           ep_w4, ep_b4, ep_w5, ep_b5):
    src = edge_index[0]
    dst = edge_index[1]
    r2 = lambda v: v.reshape(1, -1)
    ep = e.reshape(EQ, 128)

    esum, esq = _stats(ep)
    root, xsc, xsh, esc, esh = _prep(
        x, esum, esq, r2(bn_node_gamma), r2(bn_node_beta),
        r2(bn_edge_gamma), r2(bn_edge_beta), conv_root, r2(conv_bias))

    xsp = _gather_x()(x, src).reshape(EQ, 128)

    rep = jnp.repeat(jnp.eye(NF, dtype=jnp.float32), OUT, axis=1)
    w1b = _blk(esc.reshape(EF, 1) * nn_w1).astype(jnp.bfloat16)
    b1b = jnp.tile(esh @ nn_w1 + nn_b1.reshape(1, IN), (1, 8))
    w2b = _blk(nn_w2).astype(jnp.bfloat16)
    b2b = jnp.tile(nn_b2.reshape(1, IN * OUT), (1, 8))
    rxb = _blk(xsc.reshape(NF, 1) * rep).astype(jnp.bfloat16)
    shx = jnp.tile(xsh @ rep, (1, 8))
    msga, msgb = _msg(ep, xsp, w1b, b1b, w2b, b2b, rxb, shx)

    dst8 = dst.reshape(EQ, 8)
    dsta = dst8[:, 0:4].reshape(-1)
    dstb = dst8[:, 4:8].reshape(-1)
    zeros = jnp.zeros((N, OUT), jnp.float32)
    partials = _scatter_msg()(msga.reshape(EH, OUT), msgb.reshape(EH, OUT),
                              dsta, dstb, zeros)
    tbl = _combine(partials[0], partials[1], root,
                   ep_w1[0:OUT], ep_w1[OUT:2 * OUT])

    gs, gd = _gather_n()(tbl, src, dst)
    return _mlp(e, gs, gd, esc, esh, ep_w1[2 * OUT:2 * OUT + EF], r2(ep_b1),
                ep_w2, r2(ep_b2), ep_w3, r2(ep_b3), ep_w4, r2(ep_b4),
                ep_w5, r2(ep_b5))


# speedup vs baseline: 1.0458x; 1.0012x over previous
import functools

import jax
import jax.numpy as jnp
from jax import lax
from jax.experimental import pallas as pl
from jax.experimental.pallas import tpu as pltpu
from jax.experimental.pallas import tpu_sc as plsc

N = 10000
E = 320000
NF = 16
EF = 16
IN = NF
OUT = 2 * NF
LEAK = 0.1
EPS = 1e-5

NC = 2
NS = 16
NW = NC * NS
EW = E // NW

@functools.lru_cache(maxsize=None)
def _sc_mesh():
    return plsc.VectorSubcoreMesh(core_axis_name="c", subcore_axis_name="s",
                                  num_cores=NC, num_subcores=NS)


def _lrelu(v):
    return jnp.where(v >= 0, v, LEAK * v)


EQ = E // 8
_BE = 1600


def _fold16(s):
    for w in (64, 32, 16):
        s = s[:, 0:w] + s[:, w:2 * w]
    return s


def _stats_body(e_ref, sum_ref, sq_ref):
    i = pl.program_id(0)
    eb = e_ref[...]
    s = _fold16(jnp.sum(eb, axis=0, keepdims=True))
    q = _fold16(jnp.sum(eb * eb, axis=0, keepdims=True))

    @pl.when(i == 0)
    def _():
        sum_ref[...] = s
        sq_ref[...] = q

    @pl.when(i > 0)
    def _():
        sum_ref[...] += s
        sq_ref[...] += q


def _stats(ep):
    return pl.pallas_call(
        _stats_body,
        grid=(EQ // _BE,),
        in_specs=[pl.BlockSpec((_BE, 128), lambda i: (i, 0))],
        out_specs=[pl.BlockSpec((1, EF), lambda i: (0, 0)),
                   pl.BlockSpec((1, EF), lambda i: (0, 0))],
        out_shape=[jax.ShapeDtypeStruct((1, EF), jnp.float32),
                   jax.ShapeDtypeStruct((1, EF), jnp.float32)],
    )(ep)


def _prep_body(x_ref, esum_ref, esq_ref, gn_ref, bn_ref, ge_ref, be_ref,
               rw_ref, rb_ref,
               root_ref, xsc_ref, xsh_ref, esc_ref, esh_ref):
    x = x_ref[...]
    xm = jnp.mean(x, axis=0, keepdims=True)
    xv = jnp.mean(x * x, axis=0, keepdims=True) - xm * xm
    xsc = gn_ref[...] * lax.rsqrt(xv + EPS)
    xsh = bn_ref[...] - xm * xsc
    em = esum_ref[...] * (1.0 / E)
    ev = esq_ref[...] * (1.0 / E) - em * em
    esc = ge_ref[...] * lax.rsqrt(ev + EPS)
    esh = be_ref[...] - em * esc
    xb = x * xsc + xsh
    root_ref[...] = (jnp.dot(xb, rw_ref[...], preferred_element_type=jnp.float32)
                     + rb_ref[...])
    xsc_ref[...] = xsc
    xsh_ref[...] = xsh
    esc_ref[...] = esc
    esh_ref[...] = esh


def _prep(x, esum, esq, gn, bn, ge, be, rw, rb):
    v16 = jax.ShapeDtypeStruct((1, NF), jnp.float32)
    return pl.pallas_call(
        _prep_body,
        out_shape=[jax.ShapeDtypeStruct((N, OUT), jnp.float32), v16, v16, v16, v16],
    )(x, esum, esq, gn, bn, ge, be, rw, rb)


_C2 = 2000


def _gather_x_body(x_hbm, src_hbm, out_hbm, idx_v, rows_v, sem):
    wid = lax.axis_index("s") * NC + lax.axis_index("c")
    base = wid * EW

    def step(j, carry):
        off = pl.multiple_of(base + j * _C2, 8)
        pltpu.sync_copy(src_hbm.at[pl.ds(off, _C2)], idx_v)
        pltpu.async_copy(x_hbm.at[idx_v], rows_v, sem).wait()
        pltpu.sync_copy(rows_v, out_hbm.at[pl.ds(off, _C2)])
        return carry

    lax.fori_loop(0, EW // _C2, step, 0)


@functools.lru_cache(maxsize=None)
def _gather_x():
    return pl.kernel(
        _gather_x_body,
        out_type=jax.ShapeDtypeStruct((E, NF), jnp.float32),
        mesh=_sc_mesh(),
        compiler_params=pltpu.CompilerParams(use_tc_tiling_on_sc=False),
        scratch_types=[pltpu.VMEM((_C2,), jnp.int32),
                       pltpu.VMEM((_C2, NF), jnp.float32),
                       pltpu.SemaphoreType.DMA])


_BQ = 400


def _msg_body(ep_ref, xsp_ref, w1b_ref, b1b_ref, w2b_ref, b2b_ref,
              rxb_ref, shx_ref, outa_ref, outb_ref):
    f32 = jnp.float32
    bf16 = jnp.bfloat16
    hP = _lrelu(jnp.dot(ep_ref[...].astype(bf16), w1b_ref[...],
                        preferred_element_type=f32)
                + b1b_ref[...])
    zP = _lrelu(jnp.dot(hP.astype(bf16), w2b_ref[...],
                        preferred_element_type=f32)
                + b2b_ref[...])
    xeP = (jnp.dot(xsp_ref[...].astype(bf16), rxb_ref[...],
                   preferred_element_type=f32)
           + shx_ref[...])
    p = xeP * zP
    groups = []
    for g in range(8):
        q = p[:, g * 512:g * 512 + 256] + p[:, g * 512 + 256:g * 512 + 512]
        for w in (128, 64, 32):
            q = q[:, 0:w] + q[:, w:2 * w]
        groups.append(q)
    outa_ref[...] = jnp.concatenate(groups[0:4], axis=1)
    outb_ref[...] = jnp.concatenate(groups[4:8], axis=1)


def _msg(ep, xsp, w1b, b1b, w2b, b2b, rxb, shx):
    c = lambda i: (0, 0)
    return pl.pallas_call(
        _msg_body,
        grid=(EQ // _BQ,),
        in_specs=[pl.BlockSpec((_BQ, 128), lambda i: (i, 0)),
                  pl.BlockSpec((_BQ, 128), lambda i: (i, 0)),
                  pl.BlockSpec((128, 128), c), pl.BlockSpec((1, 128), c),
                  pl.BlockSpec((128, 8 * IN * OUT), c),
                  pl.BlockSpec((1, 8 * IN * OUT), c),
                  pl.BlockSpec((128, 8 * IN * OUT), c),
                  pl.BlockSpec((1, 8 * IN * OUT), c)],
        out_specs=[pl.BlockSpec((_BQ, 128), lambda i: (i, 0)),
                   pl.BlockSpec((_BQ, 128), lambda i: (i, 0))],
        out_shape=[jax.ShapeDtypeStruct((EQ, 128), jnp.float32),
                   jax.ShapeDtypeStruct((EQ, 128), jnp.float32)],
    )(ep, xsp, w1b, b1b, w2b, b2b, rxb, shx)


EH = E // 2
EHW = EH // NW
_C4 = 1000


def _scatter_msg_body(ma_hbm, mb_hbm, da_hbm, db_hbm, zero_hbm, out_hbm,
                      idx_v, upd_v, acc_sh):
    cid = lax.axis_index("c")
    sid = lax.axis_index("s")

    @pl.when(sid == 0)
    def _():
        pltpu.sync_copy(zero_hbm, acc_sh)

    plsc.subcore_barrier()
    base = (sid * NC + cid) * EHW

    def step(j, carry):
        off = pl.multiple_of(base + j * _C4, 8)
        pltpu.sync_copy(da_hbm.at[pl.ds(off, _C4)], idx_v)
        pltpu.sync_copy(ma_hbm.at[pl.ds(off, _C4)], upd_v)
        pltpu.sync_copy(upd_v, acc_sh.at[idx_v], add=True)
        pltpu.sync_copy(db_hbm.at[pl.ds(off, _C4)], idx_v)
        pltpu.sync_copy(mb_hbm.at[pl.ds(off, _C4)], upd_v)
        pltpu.sync_copy(upd_v, acc_sh.at[idx_v], add=True)
        return carry

    lax.fori_loop(0, EHW // _C4, step, 0)
    plsc.subcore_barrier()

    @pl.when(sid == 0)
    def _():
        pltpu.sync_copy(acc_sh, out_hbm.at[cid])


@functools.lru_cache(maxsize=None)
def _scatter_msg():
    return pl.kernel(
        _scatter_msg_body,
        out_type=jax.ShapeDtypeStruct((NC, N, OUT), jnp.float32),
        mesh=_sc_mesh(),
        compiler_params=pltpu.CompilerParams(use_tc_tiling_on_sc=False),
        scratch_types=[pltpu.VMEM((_C4,), jnp.int32),
                       pltpu.VMEM((_C4, OUT), jnp.float32),
                       pltpu.VMEM_SHARED((N, OUT), jnp.float32)])


def _blk(m):
    return jnp.kron(jnp.eye(8, dtype=m.dtype), m)


def _combine_body(p0_ref, p1_ref, root_ref, w1a_ref, w1b_ref, out_ref):
    f32 = jnp.float32
    node = p0_ref[...] + p1_ref[...] + root_ref[...]
    a = jnp.dot(node, w1a_ref[...], preferred_element_type=f32)
    b = jnp.dot(node, w1b_ref[...], preferred_element_type=f32)
    out_ref[...] = jnp.concatenate([a, b], axis=1)


def _combine(p0, p1, root, w1a, w1b):
    return pl.pallas_call(
        _combine_body,
        out_shape=jax.ShapeDtypeStruct((N, 128), jnp.float32),
    )(p0, p1, root, w1a, w1b)


_C6 = 1000


def _gather_n_body(nodes_hbm, src_hbm, dst_hbm, osrc_hbm, odst_hbm,
                   idx_s, idx_d, rows_v, sem):
    wid = lax.axis_index("s") * NC + lax.axis_index("c")
    base = wid * (EW // 2)

    def step(j, carry):
        off = pl.multiple_of(base + j * _C6, 8)
        pltpu.sync_copy(src_hbm.at[pl.ds(off, _C6)], idx_s)
        pltpu.sync_copy(dst_hbm.at[pl.ds(off, _C6)], idx_d)
        pltpu.async_copy(nodes_hbm.at[idx_s], rows_v, sem).wait()
        pltpu.sync_copy(rows_v, osrc_hbm.at[pl.ds(off, _C6)])
        pltpu.async_copy(nodes_hbm.at[idx_d], rows_v, sem).wait()
        pltpu.sync_copy(rows_v, odst_hbm.at[pl.ds(off, _C6)])
        return carry

    lax.fori_loop(0, EW // 2 // _C6, step, 0)


@functools.lru_cache(maxsize=None)
def _gather_n():
    return pl.kernel(
        _gather_n_body,
        out_type=(jax.ShapeDtypeStruct((E // 2, 128), jnp.float32),
                  jax.ShapeDtypeStruct((E // 2, 128), jnp.float32)),
        mesh=_sc_mesh(),
        compiler_params=pltpu.CompilerParams(use_tc_tiling_on_sc=False),
        scratch_types=[pltpu.VMEM((_C6,), jnp.int32),
                       pltpu.VMEM((_C6,), jnp.int32),
                       pltpu.VMEM((_C6, 128), jnp.float32),
                       pltpu.SemaphoreType.DMA])


_B7 = 3200


def _mlp_body(e_ref, gs_ref, gd_ref, esc_ref, esh_ref,
              w1c_ref, b1_ref, w2_ref, b2_ref, w3_ref, b3_ref,
              w4_ref, b4_ref, w5_ref, b5_ref, out_ref):
    f32 = jnp.float32
    eb = e_ref[...] * esc_ref[...] + esh_ref[...]
    z = (gs_ref[:, 0:64] + gd_ref[:, 64:128]
         + jnp.dot(eb, w1c_ref[...], preferred_element_type=f32)
         + b1_ref[...])
    z = _lrelu(z)
    z = _lrelu(jnp.dot(z, w2_ref[...], preferred_element_type=f32) + b2_ref[...])
    z = _lrelu(jnp.dot(z, w3_ref[...], preferred_element_type=f32) + b3_ref[...])
    z = _lrelu(jnp.dot(z, w4_ref[...], preferred_element_type=f32) + b4_ref[...])
    out_ref[...] = jnp.dot(z, w5_ref[...], preferred_element_type=f32) + b5_ref[...]


def _mlp(e, gs, gd, esc, esh, w1c, b1, w2, b2, w3, b3, w4, b4, w5, b5, half):
    c = lambda i: (0, 0)
    hoff = half * (E // 2 // _B7)
    return pl.pallas_call(
        _mlp_body,
        grid=(E // 2 // _B7,),
        in_specs=[pl.BlockSpec((_B7, EF), lambda i: (i + hoff, 0)),
                  pl.BlockSpec((_B7, 128), lambda i: (i, 0)),
                  pl.BlockSpec((_B7, 128), lambda i: (i, 0)),
                  pl.BlockSpec((1, EF), c), pl.BlockSpec((1, EF), c),
                  pl.BlockSpec((EF, 64), c), pl.BlockSpec((1, 64), c),
                  pl.BlockSpec((64, 32), c), pl.BlockSpec((1, 32), c),
                  pl.BlockSpec((32, 16), c), pl.BlockSpec((1, 16), c),
                  pl.BlockSpec((16, 8), c), pl.BlockSpec((1, 8), c),
                  pl.BlockSpec((8, 2), c), pl.BlockSpec((1, 2), c)],
        out_specs=pl.BlockSpec((_B7, 2), lambda i: (i, 0)),
        out_shape=jax.ShapeDtypeStruct((E // 2, 2), jnp.float32),
    )(e, gs, gd, esc, esh, w1c, b1, w2, b2, w3, b3, w4, b4, w5, b5)


def kernel(x, edge_index, e, xbatch, bn_node_gamma, bn_node_beta,
           bn_edge_gamma, bn_edge_beta, nn_w1, nn_b1, nn_w2, nn_b2,
           conv_root, conv_bias, ep_w1, ep_b1, ep_w2, ep_b2, ep_w3, ep_b3,
           ep_w4, ep_b4, ep_w5, ep_b5):
    src = edge_index[0]
    dst = edge_index[1]
    r2 = lambda v: v.reshape(1, -1)
    ep = e.reshape(EQ, 128)

    esum, esq = _stats(ep)
    root, xsc, xsh, esc, esh = _prep(
        x, esum, esq, r2(bn_node_gamma), r2(bn_node_beta),
        r2(bn_edge_gamma), r2(bn_edge_beta), conv_root, r2(conv_bias))

    xsp = _gather_x()(x, src).reshape(EQ, 128)

    rep = jnp.repeat(jnp.eye(NF, dtype=jnp.float32), OUT, axis=1)
    w1b = _blk(esc.reshape(EF, 1) * nn_w1).astype(jnp.bfloat16)
    b1b = jnp.tile(esh @ nn_w1 + nn_b1.reshape(1, IN), (1, 8))
    w2b = _blk(nn_w2).astype(jnp.bfloat16)
    b2b = jnp.tile(nn_b2.reshape(1, IN * OUT), (1, 8))
    rxb = _blk(xsc.reshape(NF, 1) * rep).astype(jnp.bfloat16)
    shx = jnp.tile(xsh @ rep, (1, 8))
    msga, msgb = _msg(ep, xsp, w1b, b1b, w2b, b2b, rxb, shx)

    dst8 = dst.reshape(EQ, 8)
    dsta = dst8[:, 0:4].reshape(-1)
    dstb = dst8[:, 4:8].reshape(-1)
    zeros = jnp.zeros((N, OUT), jnp.float32)
    partials = _scatter_msg()(msga.reshape(EH, OUT), msgb.reshape(EH, OUT),
                              dsta, dstb, zeros)
    tbl = _combine(partials[0], partials[1], root,
                   ep_w1[0:OUT], ep_w1[OUT:2 * OUT])

    mlp_w = (ep_w1[2 * OUT:2 * OUT + EF], r2(ep_b1), ep_w2, r2(ep_b2),
             ep_w3, r2(ep_b3), ep_w4, r2(ep_b4), ep_w5, r2(ep_b5))
    outs = []
    for h in range(2):
        sl = slice(h * (E // 2), (h + 1) * (E // 2))
        gs, gd = _gather_n()(tbl, src[sl], dst[sl])
        outs.append(_mlp(e, gs, gd, esc, esh, *mlp_w, half=h))
    return jnp.concatenate(outs, axis=0)


# speedup vs baseline: 1.0508x; 1.0048x over previous
import functools

import jax
import jax.numpy as jnp
from jax import lax
from jax.experimental import pallas as pl
from jax.experimental.pallas import tpu as pltpu
from jax.experimental.pallas import tpu_sc as plsc

N = 10000
E = 320000
NF = 16
EF = 16
IN = NF
OUT = 2 * NF
LEAK = 0.1
EPS = 1e-5

NC = 2
NS = 16
NW = NC * NS
EW = E // NW

@functools.lru_cache(maxsize=None)
def _sc_mesh():
    return plsc.VectorSubcoreMesh(core_axis_name="c", subcore_axis_name="s",
                                  num_cores=NC, num_subcores=NS)


def _lrelu(v):
    return jnp.where(v >= 0, v, LEAK * v)


EQ = E // 8
_BE = 1600


def _fold16(s):
    for w in (64, 32, 16):
        s = s[:, 0:w] + s[:, w:2 * w]
    return s


def _stats_body(e_ref, sum_ref, sq_ref):
    i = pl.program_id(0)
    eb = e_ref[...]
    s = _fold16(jnp.sum(eb, axis=0, keepdims=True))
    q = _fold16(jnp.sum(eb * eb, axis=0, keepdims=True))

    @pl.when(i == 0)
    def _():
        sum_ref[...] = s
        sq_ref[...] = q

    @pl.when(i > 0)
    def _():
        sum_ref[...] += s
        sq_ref[...] += q


def _stats(ep):
    return pl.pallas_call(
        _stats_body,
        grid=(EQ // _BE,),
        in_specs=[pl.BlockSpec((_BE, 128), lambda i: (i, 0))],
        out_specs=[pl.BlockSpec((1, EF), lambda i: (0, 0)),
                   pl.BlockSpec((1, EF), lambda i: (0, 0))],
        out_shape=[jax.ShapeDtypeStruct((1, EF), jnp.float32),
                   jax.ShapeDtypeStruct((1, EF), jnp.float32)],
    )(ep)


def _prep_body(x_ref, esum_ref, esq_ref, gn_ref, bn_ref, ge_ref, be_ref,
               rw_ref, rb_ref,
               root_ref, xsc_ref, xsh_ref, esc_ref, esh_ref):
    x = x_ref[...]
    xm = jnp.mean(x, axis=0, keepdims=True)
    xv = jnp.mean(x * x, axis=0, keepdims=True) - xm * xm
    xsc = gn_ref[...] * lax.rsqrt(xv + EPS)
    xsh = bn_ref[...] - xm * xsc
    em = esum_ref[...] * (1.0 / E)
    ev = esq_ref[...] * (1.0 / E) - em * em
    esc = ge_ref[...] * lax.rsqrt(ev + EPS)
    esh = be_ref[...] - em * esc
    xb = x * xsc + xsh
    root_ref[...] = (jnp.dot(xb, rw_ref[...], preferred_element_type=jnp.float32)
                     + rb_ref[...])
    xsc_ref[...] = xsc
    xsh_ref[...] = xsh
    esc_ref[...] = esc
    esh_ref[...] = esh


def _prep(x, esum, esq, gn, bn, ge, be, rw, rb):
    v16 = jax.ShapeDtypeStruct((1, NF), jnp.float32)
    return pl.pallas_call(
        _prep_body,
        out_shape=[jax.ShapeDtypeStruct((N, OUT), jnp.float32), v16, v16, v16, v16],
    )(x, esum, esq, gn, bn, ge, be, rw, rb)


_C2 = 2000


def _gather_x_body(x_hbm, src_hbm, out_hbm, idx_v, rows_v, sem):
    wid = lax.axis_index("s") * NC + lax.axis_index("c")
    base = wid * EW

    def step(j, carry):
        off = pl.multiple_of(base + j * _C2, 8)
        pltpu.sync_copy(src_hbm.at[pl.ds(off, _C2)], idx_v)
        pltpu.async_copy(x_hbm.at[idx_v], rows_v, sem).wait()
        pltpu.sync_copy(rows_v, out_hbm.at[pl.ds(off, _C2)])
        return carry

    lax.fori_loop(0, EW // _C2, step, 0)


@functools.lru_cache(maxsize=None)
def _gather_x():
    return pl.kernel(
        _gather_x_body,
        out_type=jax.ShapeDtypeStruct((E, NF), jnp.float32),
        mesh=_sc_mesh(),
        compiler_params=pltpu.CompilerParams(use_tc_tiling_on_sc=False),
        scratch_types=[pltpu.VMEM((_C2,), jnp.int32),
                       pltpu.VMEM((_C2, NF), jnp.float32),
                       pltpu.SemaphoreType.DMA])


_BQ = 800


def _msg_body(ep_ref, xsp_ref, w1b_ref, b1b_ref, w2b_ref, b2b_ref,
              rxb_ref, shx_ref, outa_ref, outb_ref):
    f32 = jnp.float32
    bf16 = jnp.bfloat16
    hP = _lrelu(jnp.dot(ep_ref[...].astype(bf16), w1b_ref[...],
                        preferred_element_type=f32)
                + b1b_ref[...])
    zP = _lrelu(jnp.dot(hP.astype(bf16), w2b_ref[...],
                        preferred_element_type=f32)
                + b2b_ref[...])
    xeP = (jnp.dot(xsp_ref[...].astype(bf16), rxb_ref[...],
                   preferred_element_type=f32)
           + shx_ref[...])
    p = xeP * zP
    groups = []
    for g in range(8):
        q = p[:, g * 512:g * 512 + 256] + p[:, g * 512 + 256:g * 512 + 512]
        for w in (128, 64, 32):
            q = q[:, 0:w] + q[:, w:2 * w]
        groups.append(q)
    outa_ref[...] = jnp.concatenate(groups[0:4], axis=1)
    outb_ref[...] = jnp.concatenate(groups[4:8], axis=1)


def _msg(ep, xsp, w1b, b1b, w2b, b2b, rxb, shx):
    c = lambda i: (0, 0)
    return pl.pallas_call(
        _msg_body,
        grid=(EQ // _BQ,),
        in_specs=[pl.BlockSpec((_BQ, 128), lambda i: (i, 0)),
                  pl.BlockSpec((_BQ, 128), lambda i: (i, 0)),
                  pl.BlockSpec((128, 128), c), pl.BlockSpec((1, 128), c),
                  pl.BlockSpec((128, 8 * IN * OUT), c),
                  pl.BlockSpec((1, 8 * IN * OUT), c),
                  pl.BlockSpec((128, 8 * IN * OUT), c),
                  pl.BlockSpec((1, 8 * IN * OUT), c)],
        out_specs=[pl.BlockSpec((_BQ, 128), lambda i: (i, 0)),
                   pl.BlockSpec((_BQ, 128), lambda i: (i, 0))],
        out_shape=[jax.ShapeDtypeStruct((EQ, 128), jnp.float32),
                   jax.ShapeDtypeStruct((EQ, 128), jnp.float32)],
    )(ep, xsp, w1b, b1b, w2b, b2b, rxb, shx)


EH = E // 2
EHW = EH // NW
_C4 = 1000


def _scatter_msg_body(ma_hbm, mb_hbm, da_hbm, db_hbm, zero_hbm, out_hbm,
                      idx_v, upd_v, acc_sh):
    cid = lax.axis_index("c")
    sid = lax.axis_index("s")

    @pl.when(sid == 0)
    def _():
        pltpu.sync_copy(zero_hbm, acc_sh)

    plsc.subcore_barrier()
    base = (sid * NC + cid) * EHW

    def step(j, carry):
        off = pl.multiple_of(base + j * _C4, 8)
        pltpu.sync_copy(da_hbm.at[pl.ds(off, _C4)], idx_v)
        pltpu.sync_copy(ma_hbm.at[pl.ds(off, _C4)], upd_v)
        pltpu.sync_copy(upd_v, acc_sh.at[idx_v], add=True)
        pltpu.sync_copy(db_hbm.at[pl.ds(off, _C4)], idx_v)
        pltpu.sync_copy(mb_hbm.at[pl.ds(off, _C4)], upd_v)
        pltpu.sync_copy(upd_v, acc_sh.at[idx_v], add=True)
        return carry

    lax.fori_loop(0, EHW // _C4, step, 0)
    plsc.subcore_barrier()

    @pl.when(sid == 0)
    def _():
        pltpu.sync_copy(acc_sh, out_hbm.at[cid])


@functools.lru_cache(maxsize=None)
def _scatter_msg():
    return pl.kernel(
        _scatter_msg_body,
        out_type=jax.ShapeDtypeStruct((NC, N, OUT), jnp.float32),
        mesh=_sc_mesh(),
        compiler_params=pltpu.CompilerParams(use_tc_tiling_on_sc=False),
        scratch_types=[pltpu.VMEM((_C4,), jnp.int32),
                       pltpu.VMEM((_C4, OUT), jnp.float32),
                       pltpu.VMEM_SHARED((N, OUT), jnp.float32)])


def _blk(m):
    return jnp.kron(jnp.eye(8, dtype=m.dtype), m)


def _combine_body(p0_ref, p1_ref, root_ref, w1a_ref, w1b_ref, out_ref):
    f32 = jnp.float32
    node = p0_ref[...] + p1_ref[...] + root_ref[...]
    a = jnp.dot(node, w1a_ref[...], preferred_element_type=f32)
    b = jnp.dot(node, w1b_ref[...], preferred_element_type=f32)
    out_ref[...] = jnp.concatenate([a, b], axis=1)


def _combine(p0, p1, root, w1a, w1b):
    return pl.pallas_call(
        _combine_body,
        out_shape=jax.ShapeDtypeStruct((N, 128), jnp.float32),
    )(p0, p1, root, w1a, w1b)


_C6 = 1000


def _gather_n_body(nodes_hbm, src_hbm, dst_hbm, osrc_hbm, odst_hbm,
                   idx_s, idx_d, rows_v, sem):
    wid = lax.axis_index("s") * NC + lax.axis_index("c")
    base = wid * (EW // 2)

    def step(j, carry):
        off = pl.multiple_of(base + j * _C6, 8)
        pltpu.sync_copy(src_hbm.at[pl.ds(off, _C6)], idx_s)
        pltpu.sync_copy(dst_hbm.at[pl.ds(off, _C6)], idx_d)
        pltpu.async_copy(nodes_hbm.at[idx_s], rows_v, sem).wait()
        pltpu.sync_copy(rows_v, osrc_hbm.at[pl.ds(off, _C6)])
        pltpu.async_copy(nodes_hbm.at[idx_d], rows_v, sem).wait()
        pltpu.sync_copy(rows_v, odst_hbm.at[pl.ds(off, _C6)])
        return carry

    lax.fori_loop(0, EW // 2 // _C6, step, 0)


@functools.lru_cache(maxsize=None)
def _gather_n():
    return pl.kernel(
        _gather_n_body,
        out_type=(jax.ShapeDtypeStruct((E // 2, 128), jnp.float32),
                  jax.ShapeDtypeStruct((E // 2, 128), jnp.float32)),
        mesh=_sc_mesh(),
        compiler_params=pltpu.CompilerParams(use_tc_tiling_on_sc=False),
        scratch_types=[pltpu.VMEM((_C6,), jnp.int32),
                       pltpu.VMEM((_C6,), jnp.int32),
                       pltpu.VMEM((_C6, 128), jnp.float32),
                       pltpu.SemaphoreType.DMA])


_B7 = 3200


def _mlp_body(e_ref, gs_ref, gd_ref, esc_ref, esh_ref,
              w1c_ref, b1_ref, w2_ref, b2_ref, w3_ref, b3_ref,
              w4_ref, b4_ref, w5_ref, b5_ref, out_ref):
    f32 = jnp.float32
    eb = e_ref[...] * esc_ref[...] + esh_ref[...]
    z = (gs_ref[:, 0:64] + gd_ref[:, 64:128]
         + jnp.dot(eb, w1c_ref[...], preferred_element_type=f32)
         + b1_ref[...])
    z = _lrelu(z)
    z = _lrelu(jnp.dot(z, w2_ref[...], preferred_element_type=f32) + b2_ref[...])
    z = _lrelu(jnp.dot(z, w3_ref[...], preferred_element_type=f32) + b3_ref[...])
    z = _lrelu(jnp.dot(z, w4_ref[...], preferred_element_type=f32) + b4_ref[...])
    out_ref[...] = jnp.dot(z, w5_ref[...], preferred_element_type=f32) + b5_ref[...]


def _mlp(e, gs, gd, esc, esh, w1c, b1, w2, b2, w3, b3, w4, b4, w5, b5, half):
    c = lambda i: (0, 0)
    hoff = half * (E // 2 // _B7)
    return pl.pallas_call(
        _mlp_body,
        grid=(E // 2 // _B7,),
        in_specs=[pl.BlockSpec((_B7, EF), lambda i: (i + hoff, 0)),
                  pl.BlockSpec((_B7, 128), lambda i: (i, 0)),
                  pl.BlockSpec((_B7, 128), lambda i: (i, 0)),
                  pl.BlockSpec((1, EF), c), pl.BlockSpec((1, EF), c),
                  pl.BlockSpec((EF, 64), c), pl.BlockSpec((1, 64), c),
                  pl.BlockSpec((64, 32), c), pl.BlockSpec((1, 32), c),
                  pl.BlockSpec((32, 16), c), pl.BlockSpec((1, 16), c),
                  pl.BlockSpec((16, 8), c), pl.BlockSpec((1, 8), c),
                  pl.BlockSpec((8, 2), c), pl.BlockSpec((1, 2), c)],
        out_specs=pl.BlockSpec((_B7, 2), lambda i: (i, 0)),
        out_shape=jax.ShapeDtypeStruct((E // 2, 2), jnp.float32),
    )(e, gs, gd, esc, esh, w1c, b1, w2, b2, w3, b3, w4, b4, w5, b5)


def kernel(x, edge_index, e, xbatch, bn_node_gamma, bn_node_beta,
           bn_edge_gamma, bn_edge_beta, nn_w1, nn_b1, nn_w2, nn_b2,
           conv_root, conv_bias, ep_w1, ep_b1, ep_w2, ep_b2, ep_w3, ep_b3,
           ep_w4, ep_b4, ep_w5, ep_b5):
    src = edge_index[0]
    dst = edge_index[1]
    r2 = lambda v: v.reshape(1, -1)
    ep = e.reshape(EQ, 128)

    esum, esq = _stats(ep)
    root, xsc, xsh, esc, esh = _prep(
        x, esum, esq, r2(bn_node_gamma), r2(bn_node_beta),
        r2(bn_edge_gamma), r2(bn_edge_beta), conv_root, r2(conv_bias))

    xsp = _gather_x()(x, src).reshape(EQ, 128)

    rep = jnp.repeat(jnp.eye(NF, dtype=jnp.float32), OUT, axis=1)
    w1b = _blk(esc.reshape(EF, 1) * nn_w1).astype(jnp.bfloat16)
    b1b = jnp.tile(esh @ nn_w1 + nn_b1.reshape(1, IN), (1, 8))
    w2b = _blk(nn_w2).astype(jnp.bfloat16)
    b2b = jnp.tile(nn_b2.reshape(1, IN * OUT), (1, 8))
    rxb = _blk(xsc.reshape(NF, 1) * rep).astype(jnp.bfloat16)
    shx = jnp.tile(xsh @ rep, (1, 8))
    msga, msgb = _msg(ep, xsp, w1b, b1b, w2b, b2b, rxb, shx)

    dst8 = dst.reshape(EQ, 8)
    dsta = dst8[:, 0:4].reshape(-1)
    dstb = dst8[:, 4:8].reshape(-1)
    zeros = jnp.zeros((N, OUT), jnp.float32)
    partials = _scatter_msg()(msga.reshape(EH, OUT), msgb.reshape(EH, OUT),
                              dsta, dstb, zeros)
    tbl = _combine(partials[0], partials[1], root,
                   ep_w1[0:OUT], ep_w1[OUT:2 * OUT])

    mlp_w = (ep_w1[2 * OUT:2 * OUT + EF], r2(ep_b1), ep_w2, r2(ep_b2),
             ep_w3, r2(ep_b3), ep_w4, r2(ep_b4), ep_w5, r2(ep_b5))
    outs = []
    for h in range(2):
        sl = slice(h * (E // 2), (h + 1) * (E // 2))
        gs, gd = _gather_n()(tbl, src[sl], dst[sl])
        outs.append(_mlp(e, gs, gd, esc, esh, *mlp_w, half=h))
    return jnp.concatenate(outs, axis=0)


# speedup vs baseline: 1.0510x; 1.0002x over previous
import functools

import jax
import jax.numpy as jnp
from jax import lax
from jax.experimental import pallas as pl
from jax.experimental.pallas import tpu as pltpu
from jax.experimental.pallas import tpu_sc as plsc

N = 10000
E = 320000
NF = 16
EF = 16
IN = NF
OUT = 2 * NF
LEAK = 0.1
EPS = 1e-5

NC = 2
NS = 16
NW = NC * NS
EW = E // NW

@functools.lru_cache(maxsize=None)
def _sc_mesh():
    return plsc.VectorSubcoreMesh(core_axis_name="c", subcore_axis_name="s",
                                  num_cores=NC, num_subcores=NS)


def _lrelu(v):
    return jnp.where(v >= 0, v, LEAK * v)


EQ = E // 8
_BE = 1600


def _fold16(s):
    for w in (64, 32, 16):
        s = s[:, 0:w] + s[:, w:2 * w]
    return s


def _stats_body(e_ref, sum_ref, sq_ref):
    i = pl.program_id(0)
    eb = e_ref[...]
    s = _fold16(jnp.sum(eb, axis=0, keepdims=True))
    q = _fold16(jnp.sum(eb * eb, axis=0, keepdims=True))

    @pl.when(i == 0)
    def _():
        sum_ref[...] = s
        sq_ref[...] = q

    @pl.when(i > 0)
    def _():
        sum_ref[...] += s
        sq_ref[...] += q


def _stats(ep):
    return pl.pallas_call(
        _stats_body,
        grid=(EQ // _BE,),
        in_specs=[pl.BlockSpec((_BE, 128), lambda i: (i, 0))],
        out_specs=[pl.BlockSpec((1, EF), lambda i: (0, 0)),
                   pl.BlockSpec((1, EF), lambda i: (0, 0))],
        out_shape=[jax.ShapeDtypeStruct((1, EF), jnp.float32),
                   jax.ShapeDtypeStruct((1, EF), jnp.float32)],
    )(ep)


def _prep_body(x_ref, esum_ref, esq_ref, gn_ref, bn_ref, ge_ref, be_ref,
               rw_ref, rb_ref,
               root_ref, xsc_ref, xsh_ref, esc_ref, esh_ref):
    x = x_ref[...]
    xm = jnp.mean(x, axis=0, keepdims=True)
    xv = jnp.mean(x * x, axis=0, keepdims=True) - xm * xm
    xsc = gn_ref[...] * lax.rsqrt(xv + EPS)
    xsh = bn_ref[...] - xm * xsc
    em = esum_ref[...] * (1.0 / E)
    ev = esq_ref[...] * (1.0 / E) - em * em
    esc = ge_ref[...] * lax.rsqrt(ev + EPS)
    esh = be_ref[...] - em * esc
    xb = x * xsc + xsh
    root_ref[...] = (jnp.dot(xb, rw_ref[...], preferred_element_type=jnp.float32)
                     + rb_ref[...])
    xsc_ref[...] = xsc
    xsh_ref[...] = xsh
    esc_ref[...] = esc
    esh_ref[...] = esh


def _prep(x, esum, esq, gn, bn, ge, be, rw, rb):
    v16 = jax.ShapeDtypeStruct((1, NF), jnp.float32)
    return pl.pallas_call(
        _prep_body,
        out_shape=[jax.ShapeDtypeStruct((N, OUT), jnp.float32), v16, v16, v16, v16],
    )(x, esum, esq, gn, bn, ge, be, rw, rb)


_C2 = 2000


def _gather_x_body(x_hbm, src_hbm, out_hbm, idx_v, rows_v, sem):
    wid = lax.axis_index("s") * NC + lax.axis_index("c")
    base = wid * EW

    def step(j, carry):
        off = pl.multiple_of(base + j * _C2, 8)
        pltpu.sync_copy(src_hbm.at[pl.ds(off, _C2)], idx_v)
        pltpu.async_copy(x_hbm.at[idx_v], rows_v, sem).wait()
        pltpu.sync_copy(rows_v, out_hbm.at[pl.ds(off, _C2)])
        return carry

    lax.fori_loop(0, EW // _C2, step, 0)


@functools.lru_cache(maxsize=None)
def _gather_x():
    return pl.kernel(
        _gather_x_body,
        out_type=jax.ShapeDtypeStruct((E, NF), jnp.float32),
        mesh=_sc_mesh(),
        compiler_params=pltpu.CompilerParams(use_tc_tiling_on_sc=False),
        scratch_types=[pltpu.VMEM((_C2,), jnp.int32),
                       pltpu.VMEM((_C2, NF), jnp.float32),
                       pltpu.SemaphoreType.DMA])


_BQ = 800


def _msg_body(ep_ref, xsp_ref, w1b_ref, b1b_ref, w2b_ref, b2b_ref,
              rxb_ref, shx_ref, outa_ref, outb_ref):
    f32 = jnp.float32
    bf16 = jnp.bfloat16
    hP = _lrelu(jnp.dot(ep_ref[...].astype(bf16), w1b_ref[...],
                        preferred_element_type=f32)
                + b1b_ref[...])
    zP = _lrelu(jnp.dot(hP.astype(bf16), w2b_ref[...],
                        preferred_element_type=f32)
                + b2b_ref[...])
    xeP = (jnp.dot(xsp_ref[...].astype(bf16), rxb_ref[...],
                   preferred_element_type=f32)
           + shx_ref[...])
    p = xeP * zP
    groups = []
    for g in range(8):
        q = p[:, g * 512:g * 512 + 256] + p[:, g * 512 + 256:g * 512 + 512]
        for w in (128, 64, 32):
            q = q[:, 0:w] + q[:, w:2 * w]
        groups.append(q)
    outa_ref[...] = jnp.concatenate(groups[0:4], axis=1)
    outb_ref[...] = jnp.concatenate(groups[4:8], axis=1)


def _msg(ep, xsp, w1b, b1b, w2b, b2b, rxb, shx):
    c = lambda i: (0, 0)
    return pl.pallas_call(
        _msg_body,
        grid=(EQ // _BQ,),
        in_specs=[pl.BlockSpec((_BQ, 128), lambda i: (i, 0)),
                  pl.BlockSpec((_BQ, 128), lambda i: (i, 0)),
                  pl.BlockSpec((128, 128), c), pl.BlockSpec((1, 128), c),
                  pl.BlockSpec((128, 8 * IN * OUT), c),
                  pl.BlockSpec((1, 8 * IN * OUT), c),
                  pl.BlockSpec((128, 8 * IN * OUT), c),
                  pl.BlockSpec((1, 8 * IN * OUT), c)],
        out_specs=[pl.BlockSpec((_BQ, 128), lambda i: (i, 0)),
                   pl.BlockSpec((_BQ, 128), lambda i: (i, 0))],
        out_shape=[jax.ShapeDtypeStruct((EQ, 128), jnp.float32),
                   jax.ShapeDtypeStruct((EQ, 128), jnp.float32)],
    )(ep, xsp, w1b, b1b, w2b, b2b, rxb, shx)


EH = E // 2
EHW = EH // NW
_C4 = 1000


def _scatter_msg_body(ma_hbm, mb_hbm, da_hbm, db_hbm, zero_hbm, out_hbm,
                      idx_v, upd_v, acc_sh):
    cid = lax.axis_index("c")
    sid = lax.axis_index("s")

    @pl.when(sid == 0)
    def _():
        pltpu.sync_copy(zero_hbm, acc_sh)

    plsc.subcore_barrier()
    base = (sid * NC + cid) * EHW

    def step(j, carry):
        off = pl.multiple_of(base + j * _C4, 8)
        pltpu.sync_copy(da_hbm.at[pl.ds(off, _C4)], idx_v)
        pltpu.sync_copy(ma_hbm.at[pl.ds(off, _C4)], upd_v)
        pltpu.sync_copy(upd_v, acc_sh.at[idx_v], add=True)
        pltpu.sync_copy(db_hbm.at[pl.ds(off, _C4)], idx_v)
        pltpu.sync_copy(mb_hbm.at[pl.ds(off, _C4)], upd_v)
        pltpu.sync_copy(upd_v, acc_sh.at[idx_v], add=True)
        return carry

    lax.fori_loop(0, EHW // _C4, step, 0)
    plsc.subcore_barrier()

    @pl.when(sid == 0)
    def _():
        pltpu.sync_copy(acc_sh, out_hbm.at[cid])


@functools.lru_cache(maxsize=None)
def _scatter_msg():
    return pl.kernel(
        _scatter_msg_body,
        out_type=jax.ShapeDtypeStruct((NC, N, OUT), jnp.float32),
        mesh=_sc_mesh(),
        compiler_params=pltpu.CompilerParams(use_tc_tiling_on_sc=False),
        scratch_types=[pltpu.VMEM((_C4,), jnp.int32),
                       pltpu.VMEM((_C4, OUT), jnp.float32),
                       pltpu.VMEM_SHARED((N, OUT), jnp.float32)])


def _blk(m):
    return jnp.kron(jnp.eye(8, dtype=m.dtype), m)


def _combine_body(p0_ref, p1_ref, root_ref, w1a_ref, w1b_ref, out_ref):
    f32 = jnp.float32
    node = p0_ref[...] + p1_ref[...] + root_ref[...]
    a = jnp.dot(node, w1a_ref[...], preferred_element_type=f32)
    b = jnp.dot(node, w1b_ref[...], preferred_element_type=f32)
    out_ref[...] = jnp.concatenate([a, b], axis=1)


def _combine(p0, p1, root, w1a, w1b):
    return pl.pallas_call(
        _combine_body,
        out_shape=jax.ShapeDtypeStruct((N, 128), jnp.float32),
    )(p0, p1, root, w1a, w1b)


_C6 = 1000


def _gather_n_body(nodes_hbm, src_hbm, dst_hbm, osrc_hbm, odst_hbm,
                   idx_s, idx_d, rows_v, sem):
    wid = lax.axis_index("s") * NC + lax.axis_index("c")
    base = wid * (EW // 2)

    def step(j, carry):
        off = pl.multiple_of(base + j * _C6, 8)
        pltpu.sync_copy(src_hbm.at[pl.ds(off, _C6)], idx_s)
        pltpu.sync_copy(dst_hbm.at[pl.ds(off, _C6)], idx_d)
        pltpu.async_copy(nodes_hbm.at[idx_s], rows_v, sem).wait()
        pltpu.sync_copy(rows_v, osrc_hbm.at[pl.ds(off, _C6)])
        pltpu.async_copy(nodes_hbm.at[idx_d], rows_v, sem).wait()
        pltpu.sync_copy(rows_v, odst_hbm.at[pl.ds(off, _C6)])
        return carry

    lax.fori_loop(0, EW // 2 // _C6, step, 0)


@functools.lru_cache(maxsize=None)
def _gather_n():
    return pl.kernel(
        _gather_n_body,
        out_type=(jax.ShapeDtypeStruct((E // 2, 128), jnp.float32),
                  jax.ShapeDtypeStruct((E // 2, 128), jnp.float32)),
        mesh=_sc_mesh(),
        compiler_params=pltpu.CompilerParams(use_tc_tiling_on_sc=False),
        scratch_types=[pltpu.VMEM((_C6,), jnp.int32),
                       pltpu.VMEM((_C6,), jnp.int32),
                       pltpu.VMEM((_C6, 128), jnp.float32),
                       pltpu.SemaphoreType.DMA])


_B7 = 6400


def _mlp_body(e_ref, gs_ref, gd_ref, esc_ref, esh_ref,
              w1c_ref, b1_ref, w2_ref, b2_ref, w3_ref, b3_ref,
              w4_ref, b4_ref, w5_ref, b5_ref, out_ref):
    f32 = jnp.float32
    eb = e_ref[...] * esc_ref[...] + esh_ref[...]
    z = (gs_ref[:, 0:64] + gd_ref[:, 64:128]
         + jnp.dot(eb, w1c_ref[...], preferred_element_type=f32)
         + b1_ref[...])
    z = _lrelu(z)
    z = _lrelu(jnp.dot(z, w2_ref[...], preferred_element_type=f32) + b2_ref[...])
    z = _lrelu(jnp.dot(z, w3_ref[...], preferred_element_type=f32) + b3_ref[...])
    z = _lrelu(jnp.dot(z, w4_ref[...], preferred_element_type=f32) + b4_ref[...])
    out_ref[...] = jnp.dot(z, w5_ref[...], preferred_element_type=f32) + b5_ref[...]


def _mlp(e, gs, gd, esc, esh, w1c, b1, w2, b2, w3, b3, w4, b4, w5, b5, half):
    c = lambda i: (0, 0)
    hoff = half * (E // 2 // _B7)
    return pl.pallas_call(
        _mlp_body,
        grid=(E // 2 // _B7,),
        in_specs=[pl.BlockSpec((_B7, EF), lambda i: (i + hoff, 0)),
                  pl.BlockSpec((_B7, 128), lambda i: (i, 0)),
                  pl.BlockSpec((_B7, 128), lambda i: (i, 0)),
                  pl.BlockSpec((1, EF), c), pl.BlockSpec((1, EF), c),
                  pl.BlockSpec((EF, 64), c), pl.BlockSpec((1, 64), c),
                  pl.BlockSpec((64, 32), c), pl.BlockSpec((1, 32), c),
                  pl.BlockSpec((32, 16), c), pl.BlockSpec((1, 16), c),
                  pl.BlockSpec((16, 8), c), pl.BlockSpec((1, 8), c),
                  pl.BlockSpec((8, 2), c), pl.BlockSpec((1, 2), c)],
        out_specs=pl.BlockSpec((_B7, 2), lambda i: (i, 0)),
        out_shape=jax.ShapeDtypeStruct((E // 2, 2), jnp.float32),
    )(e, gs, gd, esc, esh, w1c, b1, w2, b2, w3, b3, w4, b4, w5, b5)


def kernel(x, edge_index, e, xbatch, bn_node_gamma, bn_node_beta,
           bn_edge_gamma, bn_edge_beta, nn_w1, nn_b1, nn_w2, nn_b2,
           conv_root, conv_bias, ep_w1, ep_b1, ep_w2, ep_b2, ep_w3, ep_b3,
           ep_w4, ep_b4, ep_w5, ep_b5):
    src = edge_index[0]
    dst = edge_index[1]
    r2 = lambda v: v.reshape(1, -1)
    ep = e.reshape(EQ, 128)

    esum, esq = _stats(ep)
    root, xsc, xsh, esc, esh = _prep(
        x, esum, esq, r2(bn_node_gamma), r2(bn_node_beta),
        r2(bn_edge_gamma), r2(bn_edge_beta), conv_root, r2(conv_bias))

    xsp = _gather_x()(x, src).reshape(EQ, 128)

    rep = jnp.repeat(jnp.eye(NF, dtype=jnp.float32), OUT, axis=1)
    w1b = _blk(esc.reshape(EF, 1) * nn_w1).astype(jnp.bfloat16)
    b1b = jnp.tile(esh @ nn_w1 + nn_b1.reshape(1, IN), (1, 8))
    w2b = _blk(nn_w2).astype(jnp.bfloat16)
    b2b = jnp.tile(nn_b2.reshape(1, IN * OUT), (1, 8))
    rxb = _blk(xsc.reshape(NF, 1) * rep).astype(jnp.bfloat16)
    shx = jnp.tile(xsh @ rep, (1, 8))
    msga, msgb = _msg(ep, xsp, w1b, b1b, w2b, b2b, rxb, shx)

    dst8 = dst.reshape(EQ, 8)
    dsta = dst8[:, 0:4].reshape(-1)
    dstb = dst8[:, 4:8].reshape(-1)
    zeros = jnp.zeros((N, OUT), jnp.float32)
    partials = _scatter_msg()(msga.reshape(EH, OUT), msgb.reshape(EH, OUT),
                              dsta, dstb, zeros)
    tbl = _combine(partials[0], partials[1], root,
                   ep_w1[0:OUT], ep_w1[OUT:2 * OUT])

    mlp_w = (ep_w1[2 * OUT:2 * OUT + EF], r2(ep_b1), ep_w2, r2(ep_b2),
             ep_w3, r2(ep_b3), ep_w4, r2(ep_b4), ep_w5, r2(ep_b5))
    outs = []
    for h in range(2):
        sl = slice(h * (E // 2), (h + 1) * (E // 2))
        gs, gd = _gather_n()(tbl, src[sl], dst[sl])
        outs.append(_mlp(e, gs, gd, esc, esh, *mlp_w, half=h))
    return jnp.concatenate(outs, axis=0)
